# baseline (speedup 1.0000x reference)
import jax
import jax.numpy as jnp
from jax.experimental import pallas as pl

_BIN_RATE = 0.5
_THRESHOLD = 0.0


def _bin_dropout_block(mask_ref, x_ref, o_ref):
    x = x_ref[...]
    m = mask_ref[...]
    pos = x > _THRESHOLD
    o_ref[...] = jnp.where(m > 0.5, pos.astype(jnp.float32), jnp.maximum(x, 0.0))


def kernel(input):
    M, N = input.shape
    mask = jax.random.bernoulli(jax.random.key(42), _BIN_RATE, (N,))
    mask2 = mask.astype(jnp.float32)[None, :]
    block_rows = 256
    grid = (M // block_rows,)
    return pl.pallas_call(
        _bin_dropout_block,
        grid=grid,
        in_specs=[
            pl.BlockSpec((1, N), lambda i: (0, 0)),
            pl.BlockSpec((block_rows, N), lambda i: (i, 0)),
        ],
        out_specs=pl.BlockSpec((block_rows, N), lambda i: (i, 0)),
        out_shape=jax.ShapeDtypeStruct((M, N), jnp.float32),
    )(mask2, input)


# speedup vs baseline: 1.0142x; 1.0142x over previous
import jax
import jax.numpy as jnp
from jax.experimental import pallas as pl

_BIN_RATE = 0.5
_THRESHOLD = 0.0


def _bin_dropout_block(mask_ref, x_ref, o_ref):
    x = x_ref[...]
    m = mask_ref[...]
    pos = x > _THRESHOLD
    o_ref[...] = jnp.where(m > 0.5, pos.astype(jnp.float32), jnp.maximum(x, 0.0))


def kernel(input):
    M, N = input.shape
    mask = jax.random.bernoulli(jax.random.key(42), _BIN_RATE, (N,))
    mask2 = mask.astype(jnp.float32)[None, :]
    block_rows = 512
    grid = (M // block_rows,)
    return pl.pallas_call(
        _bin_dropout_block,
        grid=grid,
        in_specs=[
            pl.BlockSpec((1, N), lambda i: (0, 0)),
            pl.BlockSpec((block_rows, N), lambda i: (i, 0)),
        ],
        out_specs=pl.BlockSpec((block_rows, N), lambda i: (i, 0)),
        out_shape=jax.ShapeDtypeStruct((M, N), jnp.float32),
    )(mask2, input)


# speedup vs baseline: 1.0155x; 1.0013x over previous
import jax
import jax.numpy as jnp
from jax.experimental import pallas as pl
from jax.experimental.pallas import tpu as pltpu

_BIN_RATE = 0.5
_THRESHOLD = 0.0


def _bin_dropout_block(mask_ref, x_ref, o_ref):
    x = x_ref[...]
    m = mask_ref[...]
    pos = x > _THRESHOLD
    o_ref[...] = jnp.where(m > 0.5, pos.astype(jnp.float32), jnp.maximum(x, 0.0))


def kernel(input):
    M, N = input.shape
    mask = jax.random.bernoulli(jax.random.key(42), _BIN_RATE, (N,))
    mask2 = mask.astype(jnp.float32)[None, :]
    block_rows = 512
    grid = (M // block_rows,)
    return pl.pallas_call(
        _bin_dropout_block,
        grid=grid,
        in_specs=[
            pl.BlockSpec((1, N), lambda i: (0, 0)),
            pl.BlockSpec((block_rows, N), lambda i: (i, 0)),
        ],
        out_specs=pl.BlockSpec((block_rows, N), lambda i: (i, 0)),
        out_shape=jax.ShapeDtypeStruct((M, N), jnp.float32),
        compiler_params=pltpu.CompilerParams(
            dimension_semantics=("arbitrary",),
            vmem_limit_bytes=100 * 1024 * 1024,
        ),
    )(mask2, input)


# speedup vs baseline: 1.0271x; 1.0115x over previous
import jax
import jax.numpy as jnp
from jax.experimental import pallas as pl
from jax.experimental.pallas import tpu as pltpu

_BIN_RATE = 0.5
_THRESHOLD = 0.0


def _bin_dropout_block(mask_ref, x_ref, o_ref):
    x = x_ref[...]
    m = mask_ref[...]
    pos = x > _THRESHOLD
    o_ref[...] = jnp.where(m > 0.5, pos.astype(jnp.float32), jnp.maximum(x, 0.0))


def kernel(input):
    M, N = input.shape
    mask = jax.random.bernoulli(jax.random.key(42), _BIN_RATE, (N,))
    mask2 = mask.astype(jnp.float32)[None, :]
    block_rows = 896
    grid = (pl.cdiv(M, block_rows),)
    return pl.pallas_call(
        _bin_dropout_block,
        grid=grid,
        in_specs=[
            pl.BlockSpec((1, N), lambda i: (0, 0)),
            pl.BlockSpec((block_rows, N), lambda i: (i, 0)),
        ],
        out_specs=pl.BlockSpec((block_rows, N), lambda i: (i, 0)),
        out_shape=jax.ShapeDtypeStruct((M, N), jnp.float32),
        compiler_params=pltpu.CompilerParams(
            dimension_semantics=("arbitrary",),
            vmem_limit_bytes=100 * 1024 * 1024,
        ),
    )(mask2, input)


# speedup vs baseline: 1.0286x; 1.0015x over previous
import jax
import jax.numpy as jnp
from jax.experimental import pallas as pl
from jax.experimental.pallas import tpu as pltpu

_BIN_RATE = 0.5
_THRESHOLD = 0.0


def _bin_dropout_block(mask_ref, x_ref, o_ref):
    x = x_ref[...]
    m = mask_ref[...]
    pos = x > _THRESHOLD
    o_ref[...] = jnp.where(m > 0.5, pos.astype(jnp.float32), jnp.maximum(x, 0.0))


def kernel(input):
    M, N = input.shape
    mask = jax.random.bernoulli(jax.random.key(42), _BIN_RATE, (N,))
    mask2 = mask.astype(jnp.float32)[None, :]
    block_rows = 1008
    grid = (pl.cdiv(M, block_rows),)
    return pl.pallas_call(
        _bin_dropout_block,
        grid=grid,
        in_specs=[
            pl.BlockSpec((1, N), lambda i: (0, 0)),
            pl.BlockSpec((block_rows, N), lambda i: (i, 0)),
        ],
        out_specs=pl.BlockSpec((block_rows, N), lambda i: (i, 0)),
        out_shape=jax.ShapeDtypeStruct((M, N), jnp.float32),
        compiler_params=pltpu.CompilerParams(
            dimension_semantics=("arbitrary",),
            vmem_limit_bytes=100 * 1024 * 1024,
        ),
    )(mask2, input)
